# baseline (speedup 1.0000x reference)
import functools

import jax
import jax.numpy as jnp
from jax.experimental import pallas as pl
from jax.experimental.pallas import tpu as pltpu
from jax.experimental.pallas import tpu_sc as plsc

F32 = jnp.float32
PREC = jax.lax.Precision.DEFAULT


def _mm(a, b):
    return jax.lax.dot_general(a, b, (((1,), (0,)), ((), ())),
                               preferred_element_type=F32, precision=PREC)


_K1_DELTAS = [[(0, 0)], [(0, 0), (0, 1)], [(0, 0), (1, 0)],
              [(0, 0), (0, 1), (1, 0), (1, 1)]]


def _kenc_body(x_ref, w_ref, b_ref, w2_ref, b2_ref, o_ref):
    r = jax.lax.broadcasted_iota(jnp.int32, (3552, 1), 0)
    i = r // 59
    j = r - i * 59
    t = 0
    planes = []
    for p in range(4):
        a, b = p // 2, p % 2
        acc = None
        for ri, rj in _K1_DELTAS[p]:
            m = _mm(x_ref[0, pl.ds(ri * 59 + rj, 3552), :], w_ref[t])
            acc = m if acc is None else acc + m
            t += 1
        oh = 2 * i + a - 1
        ow = 2 * j + b - 1
        valid = ((oh >= 0) & (oh <= 112) & (ow >= 0) & (ow <= 112)
                 & (r < 3481))
        y = jnp.maximum(acc + b_ref[...], 0.0)
        planes.append(jnp.where(valid, y, 0.0).astype(jnp.bfloat16))
    acc = None
    t = 0
    for di in range(3):
        for dj in range(3):
            ph = (di % 2) * 2 + (dj % 2)
            off = (di // 2) * 59 + (dj // 2)
            v = jax.lax.slice(planes[ph], (off, 0), (off + 3488, 192))
            m = _mm(v, w2_ref[t])
            acc = m if acc is None else acc + m
            t += 1
    o_ref[0] = acc + b2_ref[...]


def _enc(x, w1, b1, w2, b2):
    xb = jnp.pad(x.transpose(0, 2, 3, 1), ((0, 0), (3, 8), (3, 8), (0, 0)))
    Q4L = xb.reshape(16, 59, 4, 59, 4, 3).transpose(0, 1, 3, 2, 4, 5)
    Q4L = Q4L.reshape(16, 3481, 48)
    Q4L = jnp.pad(Q4L, ((0, 0), (0, 135), (0, 0)))
    Q4L = Q4L.astype(jnp.bfloat16)

    blocks = []
    for p in range(4):
        a, b = p // 2, p % 2
        for ri, rj in _K1_DELTAS[p]:
            rows = []
            for u in range(4):
                for v in range(4):
                    di = u + 4 * ri - 2 * a
                    dj = v + 4 * rj - 2 * b
                    if 0 <= di <= 2 and 0 <= dj <= 2:
                        rows.append(w1[:, :, di, dj].T)
                    else:
                        rows.append(jnp.zeros((3, 192), F32))
            blocks.append(jnp.concatenate(rows, axis=0))
    W = jnp.stack(blocks, axis=0).astype(jnp.bfloat16)
    b = b1.reshape(1, 192)
    W2 = w2.transpose(2, 3, 1, 0).reshape(9, 192, 64).astype(jnp.bfloat16)
    bb2 = b2.reshape(1, 64)
    out = pl.pallas_call(
        _kenc_body,
        grid=(16,),
        in_specs=[pl.BlockSpec((1, 3616, 48), lambda n: (n, 0, 0)),
                  pl.BlockSpec((9, 48, 192), lambda n: (0, 0, 0)),
                  pl.BlockSpec((1, 192), lambda n: (0, 0)),
                  pl.BlockSpec((9, 192, 64), lambda n: (0, 0, 0)),
                  pl.BlockSpec((1, 64), lambda n: (0, 0))],
        out_specs=pl.BlockSpec((1, 3488, 64), lambda n: (n, 0, 0)),
        out_shape=jax.ShapeDtypeStruct((16, 3488, 64), F32),
    )(Q4L, W, b, W2, bb2)
    z59 = out[:, :3481].reshape(16, 59, 59, 64)[:, :57, :57, :]
    return z59


def _k3_body(x_ref, e_ref, q_ref, s_ref):
    x = x_ref[...]
    e = e_ref[...]
    e2 = jnp.sum(e * e, axis=1)
    x2 = jnp.sum(x * x, axis=1, keepdims=True)
    xe = jax.lax.dot_general(x, e, (((1,), (1,)), ((), ())),
                             preferred_element_type=F32, precision=PREC)
    d = x2 + e2[None, :] - 2.0 * xe
    idx = jnp.argmin(d, axis=1).astype(jnp.int32)
    row = pl.program_id(0) * 3264 + jax.lax.broadcasted_iota(jnp.int32, (3264, 1), 0)
    m = (row < 51984).astype(F32)
    dmin = jnp.min(d, axis=1, keepdims=True) * m
    q_ref[0, 0] = idx
    s_ref[0] = jnp.full((8, 128), jnp.sum(dmin), F32)


def _sc_gather(table, idx):
    t128 = jnp.pad(table, ((0, 0), (0, 64)))
    mesh = plsc.VectorSubcoreMesh(core_axis_name="c", subcore_axis_name="s")

    @functools.partial(
        pl.kernel, mesh=mesh,
        out_type=jax.ShapeDtypeStruct((52224, 128), F32),
        scratch_types=[pltpu.VMEM((1632,), jnp.int32),
                       pltpu.VMEM((192, 128), F32),
                       pltpu.VMEM_SHARED((512, 128), F32),
                       pltpu.SemaphoreType.DMA,
                       pltpu.SemaphoreType.DMA],
    )
    def k(table_hbm, idx_hbm, out_hbm, idx_v, rows_v, tshr, sem0, sem1):
        sems = [sem0, sem1]
        sid = jax.lax.axis_index("s")
        wid = sid * 2 + jax.lax.axis_index("c")
        base = wid * 1632

        @pl.when(sid == 0)
        def _stage():
            pltpu.sync_copy(table_hbm, tshr)

        pltpu.sync_copy(idx_hbm.at[pl.ds(base, 1632)], idx_v)
        plsc.subcore_barrier()
        cps = [None, None]
        cps[0] = pltpu.async_copy(tshr.at[idx_v.at[pl.ds(0, 96)]],
                                  rows_v.at[pl.ds(0, 96)], sems[0])
        for c in range(17):
            nx = c + 1
            if nx < 17:
                cps[nx % 2] = pltpu.async_copy(
                    tshr.at[idx_v.at[pl.ds(nx * 96, 96)]],
                    rows_v.at[pl.ds((nx % 2) * 96, 96)], sems[nx % 2])
            cps[c % 2].wait()
            pltpu.sync_copy(rows_v.at[pl.ds((c % 2) * 96, 96)],
                            out_hbm.at[pl.ds(base + c * 96, 96)])

    return k(t128, idx)


def _vq(flat_x, emb):
    xpad = jnp.pad(flat_x, ((0, 240), (0, 0)))
    idx, s = pl.pallas_call(
        _k3_body,
        grid=(16,),
        in_specs=[pl.BlockSpec((3264, 64), lambda i: (i, 0)),
                  pl.BlockSpec((512, 64), lambda i: (0, 0))],
        out_specs=[pl.BlockSpec((1, 1, 3264), lambda i: (i, 0, 0)),
                   pl.BlockSpec((1, 8, 128), lambda i: (i, 0, 0))],
        out_shape=[jax.ShapeDtypeStruct((16, 1, 3264), jnp.int32),
                   jax.ShapeDtypeStruct((16, 8, 128), F32)],
    )(xpad, emb)
    q = _sc_gather(emb, idx.reshape(52224))
    return q[:51984, :64], jnp.sum(s[:, 0, 0])


def _k45_body(q_ref, w4_ref, b4_ref, w5_ref, b5_ref, o_ref):
    b4 = b4_ref[...]

    def sl4(o):
        return q_ref[0, pl.ds(o, 3328), :]

    ee = jnp.maximum(_mm(sl4(0), w4_ref[0]) + b4, 0.0)
    eo = jnp.maximum(_mm(sl4(0), w4_ref[1]) + _mm(sl4(1), w4_ref[2]) + b4, 0.0)
    oe = jnp.maximum(_mm(sl4(0), w4_ref[3]) + _mm(sl4(57), w4_ref[4]) + b4, 0.0)
    oo = jnp.maximum(_mm(sl4(0), w4_ref[5]) + _mm(sl4(1), w4_ref[6])
                     + _mm(sl4(57), w4_ref[7]) + _mm(sl4(58), w4_ref[8]), 0.0)
    planes = [p.astype(jnp.bfloat16) for p in (ee, eo, oe, oo)]

    acc = None
    t = 0
    for pr, du in ((0, 0), (1, 0), (0, 1)):
        for pc, dv in ((0, 0), (1, 0), (0, 1)):
            pln = planes[pr * 2 + pc]
            off = du * 57 + dv
            v = jax.lax.slice(pln, (off, 0), (off + 3264, 192))
            m = _mm(v, w5_ref[t])
            acc = m if acc is None else acc + m
            t += 1
    o_ref[0] = jax.nn.sigmoid(acc + b5_ref[...])


def _dec(qn, w4, b4, w5, b5, x):
    Q = qn.reshape(16, 3249, 64)
    Q = jnp.pad(Q, ((0, 0), (0, 143), (0, 0)))
    Q = Q.astype(jnp.bfloat16)
    taps4 = [w4[:, :, 1, 1],
             w4[:, :, 1, 2], w4[:, :, 1, 0],
             w4[:, :, 2, 1], w4[:, :, 0, 1],
             w4[:, :, 2, 2], w4[:, :, 2, 0], w4[:, :, 0, 2], w4[:, :, 0, 0]]
    W4 = jnp.stack(taps4, axis=0).astype(jnp.bfloat16)
    bb4 = b4.reshape(1, 192)

    rterms = {(0, 0): [(0, 1), (1, 2)],
              (1, 0): [(1, 0), (2, 1), (3, 2)],
              (0, 1): [(3, 0)]}
    combos = []
    for pr, du in ((0, 0), (1, 0), (0, 1)):
        for pc, dv in ((0, 0), (1, 0), (0, 1)):
            blocks = []
            feed = {}
            for c, kh in rterms[(pr, du)]:
                for e, kw in rterms[(pc, dv)]:
                    feed[c * 4 + e] = (kh, kw)
            for blk in range(16):
                if blk in feed:
                    kh, kw = feed[blk]
                    blocks.append(jnp.pad(w5[:, :, kh, kw], ((0, 0), (0, 5))))
                else:
                    blocks.append(jnp.zeros((192, 8), F32))
            combos.append(jnp.concatenate(blocks, axis=1))
    W5 = jnp.stack(combos, axis=0).astype(jnp.bfloat16)
    bb5 = jnp.tile(jnp.pad(b5, (0, 5)), 16).reshape(1, 128)

    rec = pl.pallas_call(
        _k45_body,
        grid=(16,),
        in_specs=[pl.BlockSpec((1, 3392, 64), lambda n: (n, 0, 0)),
                  pl.BlockSpec((9, 64, 192), lambda n: (0, 0, 0)),
                  pl.BlockSpec((1, 192), lambda n: (0, 0)),
                  pl.BlockSpec((9, 192, 128), lambda n: (0, 0, 0)),
                  pl.BlockSpec((1, 128), lambda n: (0, 0))],
        out_specs=pl.BlockSpec((1, 3264, 128), lambda n: (n, 0, 0)),
        out_shape=jax.ShapeDtypeStruct((16, 3264, 128), F32),
    )(Q, W4, bb4, W5, bb5)

    R = rec[:, :3249].reshape(16, 57, 57, 4, 4, 8)[..., :3]
    R = R.transpose(0, 1, 3, 2, 4, 5).reshape(16, 228, 228, 3)
    x_recon = R[:, :225, :225].transpose(0, 3, 1, 2)
    return x_recon


def kernel(x, enc_w1, enc_b1, enc_w2, enc_b2, embedding,
           dec_w1, dec_b1, dec_w2, dec_b2):
    z_nhwc = _enc(x, enc_w1, enc_b1, enc_w2, enc_b2)
    z_nchw = z_nhwc.transpose(0, 3, 1, 2)
    flat_x = z_nchw.reshape(-1, 64)
    qflat, vq_sum = _vq(flat_x, embedding)
    qn = qflat.reshape(16, 64, 57, 57).transpose(0, 2, 3, 1)
    x_recon = _dec(qn, dec_w1, dec_b1, dec_w2, dec_b2, x)
    recon_loss = jnp.mean(jnp.square(x_recon - x))
    vq_loss = 1.25 * vq_sum / 3326976.0
    return (x_recon, recon_loss + vq_loss)


# speedup vs baseline: 1.0072x; 1.0072x over previous
import functools

import jax
import jax.numpy as jnp
from jax.experimental import pallas as pl
from jax.experimental.pallas import tpu as pltpu
from jax.experimental.pallas import tpu_sc as plsc

F32 = jnp.float32
PREC = jax.lax.Precision.DEFAULT


def _mm(a, b):
    return jax.lax.dot_general(a, b, (((1,), (0,)), ((), ())),
                               preferred_element_type=F32, precision=PREC)


_K1_DELTAS = [[(0, 0)], [(0, 0), (0, 1)], [(0, 0), (1, 0)],
              [(0, 0), (0, 1), (1, 0), (1, 1)]]


def _kenc_body(x_ref, w_ref, b_ref, w2_ref, b2_ref, o_ref):
    r = jax.lax.broadcasted_iota(jnp.int32, (3552, 1), 0)
    i = r // 59
    j = r - i * 59
    t = 0
    planes = []
    for p in range(4):
        a, b = p // 2, p % 2
        acc = None
        for ri, rj in _K1_DELTAS[p]:
            m = _mm(x_ref[0, pl.ds(ri * 59 + rj, 3552), :], w_ref[t])
            acc = m if acc is None else acc + m
            t += 1
        oh = 2 * i + a - 1
        ow = 2 * j + b - 1
        valid = ((oh >= 0) & (oh <= 112) & (ow >= 0) & (ow <= 112)
                 & (r < 3481))
        y = jnp.maximum(acc + b_ref[...], 0.0)
        planes.append(jnp.where(valid, y, 0.0))
    acc = None
    t = 0
    for di in range(3):
        for dj in range(3):
            ph = (di % 2) * 2 + (dj % 2)
            off = (di // 2) * 59 + (dj // 2)
            v = jax.lax.slice(planes[ph], (off, 0), (off + 3488, 192))
            m = _mm(v, w2_ref[t])
            acc = m if acc is None else acc + m
            t += 1
    o_ref[0] = acc + b2_ref[...]


def _enc(x, w1, b1, w2, b2):
    xb = jnp.pad(x.transpose(0, 2, 3, 1), ((0, 0), (3, 8), (3, 8), (0, 0)))
    Q4L = xb.reshape(16, 59, 4, 59, 4, 3).transpose(0, 1, 3, 2, 4, 5)
    Q4L = Q4L.reshape(16, 3481, 48)
    Q4L = jnp.pad(Q4L, ((0, 0), (0, 135), (0, 0)))

    blocks = []
    for p in range(4):
        a, b = p // 2, p % 2
        for ri, rj in _K1_DELTAS[p]:
            rows = []
            for u in range(4):
                for v in range(4):
                    di = u + 4 * ri - 2 * a
                    dj = v + 4 * rj - 2 * b
                    if 0 <= di <= 2 and 0 <= dj <= 2:
                        rows.append(w1[:, :, di, dj].T)
                    else:
                        rows.append(jnp.zeros((3, 192), F32))
            blocks.append(jnp.concatenate(rows, axis=0))
    W = jnp.stack(blocks, axis=0)
    b = b1.reshape(1, 192)
    W2 = w2.transpose(2, 3, 1, 0).reshape(9, 192, 64)
    bb2 = b2.reshape(1, 64)
    out = pl.pallas_call(
        _kenc_body,
        grid=(16,),
        in_specs=[pl.BlockSpec((1, 3616, 48), lambda n: (n, 0, 0)),
                  pl.BlockSpec((9, 48, 192), lambda n: (0, 0, 0)),
                  pl.BlockSpec((1, 192), lambda n: (0, 0)),
                  pl.BlockSpec((9, 192, 64), lambda n: (0, 0, 0)),
                  pl.BlockSpec((1, 64), lambda n: (0, 0))],
        out_specs=pl.BlockSpec((1, 3488, 64), lambda n: (n, 0, 0)),
        out_shape=jax.ShapeDtypeStruct((16, 3488, 64), F32),
    )(Q4L, W, b, W2, bb2)
    z59 = out[:, :3481].reshape(16, 59, 59, 64)[:, :57, :57, :]
    return z59


def _k3_body(x_ref, e_ref, q_ref, s_ref):
    x = x_ref[...]
    e = e_ref[...]
    e2 = jnp.sum(e * e, axis=1)
    x2 = jnp.sum(x * x, axis=1, keepdims=True)
    xe = jax.lax.dot_general(x, e, (((1,), (1,)), ((), ())),
                             preferred_element_type=F32, precision=PREC)
    d = x2 + e2[None, :] - 2.0 * xe
    idx = jnp.argmin(d, axis=1).astype(jnp.int32)
    row = pl.program_id(0) * 3264 + jax.lax.broadcasted_iota(jnp.int32, (3264, 1), 0)
    m = (row < 51984).astype(F32)
    dmin = jnp.min(d, axis=1, keepdims=True) * m
    q_ref[0, 0] = idx
    s_ref[0] = jnp.full((8, 128), jnp.sum(dmin), F32)


def _sc_gather(table, idx):
    t128 = jnp.pad(table, ((0, 0), (0, 64)))
    mesh = plsc.VectorSubcoreMesh(core_axis_name="c", subcore_axis_name="s")

    @functools.partial(
        pl.kernel, mesh=mesh,
        out_type=jax.ShapeDtypeStruct((52224, 128), F32),
        scratch_types=[pltpu.VMEM((1632,), jnp.int32),
                       pltpu.VMEM((192, 128), F32),
                       pltpu.VMEM_SHARED((512, 128), F32),
                       pltpu.SemaphoreType.DMA,
                       pltpu.SemaphoreType.DMA],
    )
    def k(table_hbm, idx_hbm, out_hbm, idx_v, rows_v, tshr, sem0, sem1):
        sems = [sem0, sem1]
        sid = jax.lax.axis_index("s")
        wid = sid * 2 + jax.lax.axis_index("c")
        base = wid * 1632

        @pl.when(sid == 0)
        def _stage():
            pltpu.sync_copy(table_hbm, tshr)

        pltpu.sync_copy(idx_hbm.at[pl.ds(base, 1632)], idx_v)
        plsc.subcore_barrier()
        cps = [None, None]
        cps[0] = pltpu.async_copy(tshr.at[idx_v.at[pl.ds(0, 96)]],
                                  rows_v.at[pl.ds(0, 96)], sems[0])
        for c in range(17):
            nx = c + 1
            if nx < 17:
                cps[nx % 2] = pltpu.async_copy(
                    tshr.at[idx_v.at[pl.ds(nx * 96, 96)]],
                    rows_v.at[pl.ds((nx % 2) * 96, 96)], sems[nx % 2])
            cps[c % 2].wait()
            pltpu.sync_copy(rows_v.at[pl.ds((c % 2) * 96, 96)],
                            out_hbm.at[pl.ds(base + c * 96, 96)])

    return k(t128, idx)


def _vq(flat_x, emb):
    xpad = jnp.pad(flat_x, ((0, 240), (0, 0)))
    idx, s = pl.pallas_call(
        _k3_body,
        grid=(16,),
        in_specs=[pl.BlockSpec((3264, 64), lambda i: (i, 0)),
                  pl.BlockSpec((512, 64), lambda i: (0, 0))],
        out_specs=[pl.BlockSpec((1, 1, 3264), lambda i: (i, 0, 0)),
                   pl.BlockSpec((1, 8, 128), lambda i: (i, 0, 0))],
        out_shape=[jax.ShapeDtypeStruct((16, 1, 3264), jnp.int32),
                   jax.ShapeDtypeStruct((16, 8, 128), F32)],
    )(xpad, emb)
    q = _sc_gather(emb, idx.reshape(52224))
    return q[:51984, :64], jnp.sum(s[:, 0, 0])


def _k45_body(q_ref, w4_ref, b4_ref, w5_ref, b5_ref, o_ref):
    b4 = b4_ref[...]

    def sl4(o):
        return q_ref[0, pl.ds(o, 3328), :]

    ee = jnp.maximum(_mm(sl4(0), w4_ref[0]) + b4, 0.0)
    eo = jnp.maximum(_mm(sl4(0), w4_ref[1]) + _mm(sl4(1), w4_ref[2]) + b4, 0.0)
    oe = jnp.maximum(_mm(sl4(0), w4_ref[3]) + _mm(sl4(57), w4_ref[4]) + b4, 0.0)
    oo = jnp.maximum(_mm(sl4(0), w4_ref[5]) + _mm(sl4(1), w4_ref[6])
                     + _mm(sl4(57), w4_ref[7]) + _mm(sl4(58), w4_ref[8]), 0.0)
    planes = [ee, eo, oe, oo]

    acc = None
    t = 0
    for pr, du in ((0, 0), (1, 0), (0, 1)):
        for pc, dv in ((0, 0), (1, 0), (0, 1)):
            pln = planes[pr * 2 + pc]
            off = du * 57 + dv
            v = jax.lax.slice(pln, (off, 0), (off + 3264, 192))
            m = _mm(v, w5_ref[t])
            acc = m if acc is None else acc + m
            t += 1
    o_ref[0] = jax.nn.sigmoid(acc + b5_ref[...])


def _dec(qn, w4, b4, w5, b5, x):
    Q = qn.reshape(16, 3249, 64)
    Q = jnp.pad(Q, ((0, 0), (0, 143), (0, 0)))
    taps4 = [w4[:, :, 1, 1],
             w4[:, :, 1, 2], w4[:, :, 1, 0],
             w4[:, :, 2, 1], w4[:, :, 0, 1],
             w4[:, :, 2, 2], w4[:, :, 2, 0], w4[:, :, 0, 2], w4[:, :, 0, 0]]
    W4 = jnp.stack(taps4, axis=0)
    bb4 = b4.reshape(1, 192)

    rterms = {(0, 0): [(0, 1), (1, 2)],
              (1, 0): [(1, 0), (2, 1), (3, 2)],
              (0, 1): [(3, 0)]}
    combos = []
    for pr, du in ((0, 0), (1, 0), (0, 1)):
        for pc, dv in ((0, 0), (1, 0), (0, 1)):
            blocks = []
            feed = {}
            for c, kh in rterms[(pr, du)]:
                for e, kw in rterms[(pc, dv)]:
                    feed[c * 4 + e] = (kh, kw)
            for blk in range(16):
                if blk in feed:
                    kh, kw = feed[blk]
                    blocks.append(jnp.pad(w5[:, :, kh, kw], ((0, 0), (0, 5))))
                else:
                    blocks.append(jnp.zeros((192, 8), F32))
            combos.append(jnp.concatenate(blocks, axis=1))
    W5 = jnp.stack(combos, axis=0)
    bb5 = jnp.tile(jnp.pad(b5, (0, 5)), 16).reshape(1, 128)

    rec = pl.pallas_call(
        _k45_body,
        grid=(16,),
        in_specs=[pl.BlockSpec((1, 3392, 64), lambda n: (n, 0, 0)),
                  pl.BlockSpec((9, 64, 192), lambda n: (0, 0, 0)),
                  pl.BlockSpec((1, 192), lambda n: (0, 0)),
                  pl.BlockSpec((9, 192, 128), lambda n: (0, 0, 0)),
                  pl.BlockSpec((1, 128), lambda n: (0, 0))],
        out_specs=pl.BlockSpec((1, 3264, 128), lambda n: (n, 0, 0)),
        out_shape=jax.ShapeDtypeStruct((16, 3264, 128), F32),
    )(Q, W4, bb4, W5, bb5)

    R = rec[:, :3249].reshape(16, 57, 57, 4, 4, 8)[..., :3]
    R = R.transpose(0, 1, 3, 2, 4, 5).reshape(16, 228, 228, 3)
    x_recon = R[:, :225, :225].transpose(0, 3, 1, 2)
    return x_recon


def kernel(x, enc_w1, enc_b1, enc_w2, enc_b2, embedding,
           dec_w1, dec_b1, dec_w2, dec_b2):
    z_nhwc = _enc(x, enc_w1, enc_b1, enc_w2, enc_b2)
    z_nchw = z_nhwc.transpose(0, 3, 1, 2)
    flat_x = z_nchw.reshape(-1, 64)
    qflat, vq_sum = _vq(flat_x, embedding)
    qn = qflat.reshape(16, 64, 57, 57).transpose(0, 2, 3, 1)
    x_recon = _dec(qn, dec_w1, dec_b1, dec_w2, dec_b2, x)
    recon_loss = jnp.mean(jnp.square(x_recon - x))
    vq_loss = 1.25 * vq_sum / 3326976.0
    return (x_recon, recon_loss + vq_loss)
